# baseline (speedup 1.0000x reference)
import functools

import jax
import jax.numpy as jnp
from jax import lax
from jax.experimental import pallas as pl
from jax.experimental.pallas import tpu as pltpu
from jax.experimental.pallas import tpu_sc as plsc

N = 10000
D = 128
E = 320000

NW = 32
K = 128
CHUNKS = 79
EPW = K * CHUNKS
EPAD = EPW * NW
NPAD = 10112
NROWCH = NPAD // K
NPS = NPAD // 16

CH_A = 118
CH_B = 40
MAXCH = max(CH_A, CH_B)

_mesh = plsc.VectorSubcoreMesh(core_axis_name="c", subcore_axis_name="s")


@functools.partial(
    pl.kernel,
    out_type=jax.ShapeDtypeStruct((2 * NPAD,), jnp.float32),
    mesh=_mesh,
    scratch_types=[
        pltpu.VMEM((K,), jnp.int32),
        pltpu.VMEM((K,), jnp.int32),
        pltpu.VMEM((K,), jnp.float32),
        pltpu.VMEM((NPS,), jnp.float32),
        pltpu.VMEM_SHARED((NPAD,), jnp.float32),
        pltpu.SemaphoreType.DMA,
        pltpu.SemaphoreType.DMA,
    ],
)
def _deg_kernel(dst_hbm, out_hbm, dst_v0, dst_v1, ones_v, stage_v, acc_sh,
                dsem0, dsem1):
    c = lax.axis_index("c")
    s = lax.axis_index("s")
    wid = s * 2 + c

    ones16 = jnp.ones((16,), jnp.float32)
    zeros16 = jnp.zeros((16,), jnp.float32)

    def _init(i, _):
        ones_v[pl.ds(i * 16, 16)] = ones16
        return 0

    lax.fori_loop(0, K // 16, _init, 0)

    def _zero(i, _):
        stage_v[pl.ds(i * 16, 16)] = zeros16
        return 0

    lax.fori_loop(0, NPS // 16, _zero, 0)

    pltpu.sync_copy(stage_v, acc_sh.at[pl.ds(s * NPS, NPS)])
    plsc.subcore_barrier()

    base = wid * EPW
    dbufs = ((dst_v0, dsem0), (dst_v1, dsem1))

    def _dissue(i, b):
        dstv, dsem = dbufs[b]
        pltpu.async_copy(dst_hbm.at[pl.ds(base + i * K, K)], dstv, dsem)

    def _dconsume(i, b):
        dstv, dsem = dbufs[b]
        pltpu.make_async_copy(dst_hbm.at[pl.ds(base + i * K, K)], dstv,
                              dsem).wait()
        pltpu.sync_copy(ones_v, acc_sh.at[dstv], add=True)

    _dissue(0, 0)

    def _body(t, _):
        for b in range(2):
            i = 2 * t + b

            @pl.when(i + 1 < CHUNKS)
            def _():
                _dissue(i + 1, 1 - b)

            @pl.when(i < CHUNKS)
            def _():
                _dconsume(i, b)

        return 0

    lax.fori_loop(0, (CHUNKS + 1) // 2, _body, 0)
    plsc.subcore_barrier()

    pltpu.sync_copy(acc_sh.at[pl.ds(s * NPS, NPS)], stage_v)
    pltpu.sync_copy(stage_v, out_hbm.at[pl.ds(c * NPAD + s * NPS, NPS)])


@functools.partial(
    pl.kernel,
    out_type=[jax.ShapeDtypeStruct((NPAD, D), jnp.float32),
              jax.ShapeDtypeStruct((NPAD // 2, D), jnp.float32)],
    mesh=_mesh,
    scratch_types=[
        pltpu.VMEM((K,), jnp.int32),
        pltpu.VMEM((K,), jnp.int32),
        pltpu.VMEM((K,), jnp.int32),
        pltpu.VMEM((K,), jnp.int32),
        pltpu.VMEM((K,), jnp.int32),
        pltpu.VMEM((K,), jnp.int32),
        pltpu.VMEM((K, D), jnp.float32),
        pltpu.VMEM((K, D), jnp.float32),
        pltpu.VMEM((K, D), jnp.float32),
        pltpu.VMEM_SHARED((NPAD, D), jnp.float32),
        pltpu.SemaphoreType.DMA,
        pltpu.SemaphoreType.DMA,
        pltpu.SemaphoreType.DMA,
        pltpu.SemaphoreType.DMA,
        pltpu.SemaphoreType.DMA,
        pltpu.SemaphoreType.DMA,
    ],
)
def _scatter_kernel(g_hbm, src_hbm, dst_hbm, out0_hbm, out1_hbm,
                    srcv0, srcv1, srcv2, dstv0, dstv1, dstv2,
                    rows0, rows1, rows2, acc_sh,
                    sem0, sem1, sem2, dsem0, dsem1, dsem2):
    c = lax.axis_index("c")
    s = lax.axis_index("s")

    zeros16 = jnp.zeros((16,), jnp.float32)

    def _zrow(i, _):
        for j in range(D // 16):
            rows0[i, pl.ds(j * 16, 16)] = zeros16
        return 0

    lax.fori_loop(0, K, _zrow, 0)

    nch = jnp.where(c == 0, CH_A, CH_B)
    base = jnp.where(c == 0, s * CH_A, 16 * CH_A + s * CH_B) * K

    NZT = (NROWCH + 15) // 16
    for t in range(NZT):
        cc = s + 16 * t

        @pl.when(cc < NROWCH)
        def _():
            pltpu.async_copy(rows0, acc_sh.at[pl.ds(cc * K, K)], sem0)

    for t in range(NZT):
        cc = s + 16 * t

        @pl.when(cc < NROWCH)
        def _():
            pltpu.make_async_copy(rows0, acc_sh.at[pl.ds(cc * K, K)],
                                  sem0).wait()

    plsc.subcore_barrier()

    bufs = ((rows0, sem0, srcv0, dstv0, dsem0),
            (rows1, sem1, srcv1, dstv1, dsem1),
            (rows2, sem2, srcv2, dstv2, dsem2))
    NB = len(bufs)

    def _issue_idx(i, b):
        _, _, srcv, dstv, dsem = bufs[b]
        pltpu.async_copy(src_hbm.at[pl.ds(base + i * K, K)], srcv, dsem)
        pltpu.async_copy(dst_hbm.at[pl.ds(base + i * K, K)], dstv, dsem)

    def _issue_gather(i, b):
        rows, sem, srcv, dstv, dsem = bufs[b]
        pltpu.make_async_copy(src_hbm.at[pl.ds(base + i * K, K)], srcv,
                              dsem).wait()
        pltpu.make_async_copy(dst_hbm.at[pl.ds(base + i * K, K)], dstv,
                              dsem).wait()
        pltpu.async_copy(g_hbm.at[srcv], rows, sem)

    def _consume(i, b):
        rows, sem, srcv, dstv, dsem = bufs[b]
        pltpu.make_async_copy(g_hbm.at[srcv], rows, sem).wait()
        pltpu.sync_copy(rows, acc_sh.at[dstv], add=True)

    _issue_idx(0, 0)
    _issue_idx(1, 1)
    _issue_gather(0, 0)

    def _body(t, _):
        for b in range(NB):
            i = NB * t + b

            @pl.when(i + 2 < nch)
            def _():
                _issue_idx(i + 2, (b + 2) % NB)

            @pl.when(i + 1 < nch)
            def _():
                _issue_gather(i + 1, (b + 1) % NB)

            @pl.when(i < nch)
            def _():
                _consume(i, b)

        return 0

    lax.fori_loop(0, (MAXCH + NB - 1) // NB, _body, 0)
    plsc.subcore_barrier()

    rbufs = ((rows0, sem0), (rows1, sem1), (rows2, sem2))

    def _hbm_write(t):
        cc = s + 16 * t

        @pl.when(cc < NROWCH)
        def _():
            b, wsem = rbufs[t % 3]
            pltpu.async_copy(acc_sh.at[pl.ds(cc * K, K)], b, dsem0)
            pltpu.make_async_copy(acc_sh.at[pl.ds(cc * K, K)], b,
                                  dsem0).wait()

            @pl.when(c == 0)
            def _():
                pltpu.async_copy(b, out0_hbm.at[pl.ds(cc * K, K)], wsem)

            @pl.when(c == 1)
            def _():
                def _pack(i, _):
                    for j in range(D // 16):
                        v0 = b[2 * i, pl.ds(16 * j, 16)]
                        v1 = b[2 * i + 1, pl.ds(16 * j, 16)]
                        i0 = lax.bitcast_convert_type(v0, jnp.int32) + 0x8000
                        i1 = lax.bitcast_convert_type(v1, jnp.int32) + 0x8000
                        hi = jnp.bitwise_and(i0, jnp.int32(-65536))
                        lo = lax.shift_right_logical(i1, 16)
                        b[i, pl.ds(16 * j, 16)] = lax.bitcast_convert_type(
                            jnp.bitwise_or(hi, lo), jnp.float32)
                    return 0

                lax.fori_loop(0, K // 2, _pack, 0)
                pltpu.async_copy(b.at[pl.ds(0, K // 2)],
                                 out1_hbm.at[pl.ds(cc * (K // 2), K // 2)],
                                 wsem)

    def _hbm_drain(t):
        cc = s + 16 * t

        @pl.when(cc < NROWCH)
        def _():
            b, wsem = rbufs[t % 3]

            @pl.when(c == 0)
            def _():
                pltpu.make_async_copy(
                    b, out0_hbm.at[pl.ds(cc * K, K)], wsem).wait()

            @pl.when(c == 1)
            def _():
                pltpu.make_async_copy(
                    b.at[pl.ds(0, K // 2)],
                    out1_hbm.at[pl.ds(cc * (K // 2), K // 2)], wsem).wait()

    NCT = (NROWCH + 15) // 16
    for t in range(NCT):
        if t >= 3:
            _hbm_drain(t - 3)
        _hbm_write(t)
    for t in range(max(0, NCT - 3), NCT):
        _hbm_drain(t)


_RB = 400
_GRID = N // _RB


def _mm_scale_body(x_ref, wt_ref, dinv_ref, o_ref):
    o_ref[...] = (
        jnp.dot(x_ref[...], wt_ref[...], preferred_element_type=jnp.float32)
        * dinv_ref[...]
    )


def _layer_body(g_ref, a0_ref, a1_ref, dinv_ref, b_ref, wt_ref, o_ref):
    h = (g_ref[...] + a0_ref[...] + a1_ref[...]) * dinv_ref[...] + b_ref[...]
    h = jnp.maximum(h, 0.0)
    o_ref[...] = (
        jnp.dot(h, wt_ref[...], preferred_element_type=jnp.float32)
        * dinv_ref[...]
    )


def _final_body(g_ref, a0_ref, a1_ref, dinv_ref, b_ref, o_ref):
    o_ref[...] = (
        (g_ref[...] + a0_ref[...] + a1_ref[...]) * dinv_ref[...] + b_ref[...]
    )


_row_spec = pl.BlockSpec((_RB, D), lambda i: (i, 0))
_w_spec = pl.BlockSpec((D, D), lambda i: (0, 0))
_dinv_spec = pl.BlockSpec((_RB, 1), lambda i: (i, 0))
_b_spec = pl.BlockSpec((1, D), lambda i: (0, 0))
_node_shape = jax.ShapeDtypeStruct((N, D), jnp.float32)


def _mm_scale(x, wt, dinv_col):
    return pl.pallas_call(
        _mm_scale_body,
        grid=(_GRID,),
        in_specs=[_row_spec, _w_spec, _dinv_spec],
        out_specs=_row_spec,
        out_shape=_node_shape,
    )(x, wt, dinv_col)


def _layer(g, a0, a1, dinv_col, b_row, wt):
    return pl.pallas_call(
        _layer_body,
        grid=(_GRID,),
        in_specs=[_row_spec, _row_spec, _row_spec, _dinv_spec, _b_spec,
                  _w_spec],
        out_specs=_row_spec,
        out_shape=_node_shape,
    )(g, a0, a1, dinv_col, b_row, wt)


def _final(g, a0, a1, dinv_col, b_row):
    return pl.pallas_call(
        _final_body,
        grid=(_GRID,),
        in_specs=[_row_spec, _row_spec, _row_spec, _dinv_spec, _b_spec],
        out_specs=_row_spec,
        out_shape=_node_shape,
    )(g, a0, a1, dinv_col, b_row)


def kernel(x, edge_index, W1, b1, W2, b2):
    src = edge_index[0].astype(jnp.int32)
    dst = edge_index[1].astype(jnp.int32)
    pad = EPAD - E
    src_p = jnp.concatenate([src, jnp.zeros((pad,), jnp.int32)])
    dst_p = jnp.concatenate([dst, jnp.full((pad,), N, jnp.int32)])

    degp = _deg_kernel(dst_p)
    deg = degp[:N] + degp[NPAD:NPAD + N] + 1.0
    dinv_col = lax.rsqrt(deg)[:, None]
    b1_row = b1[None, :]
    b2_row = b2[None, :]

    def _unpack_slow(ap):
        w = lax.bitcast_convert_type(ap, jnp.bfloat16)
        w = jnp.stack([w[..., 1], w[..., 0]], axis=1)
        return w.reshape(NPAD, D).astype(jnp.float32)

    g1 = _mm_scale(x, W1.T, dinv_col)
    a1f, a1s = _scatter_kernel(g1, src_p, dst_p)
    a1s = _unpack_slow(a1s)
    g2 = _layer(g1, a1f[:N], a1s[:N], dinv_col, b1_row, W2.T)
    a2f, a2s = _scatter_kernel(g2, src_p, dst_p)
    a2s = _unpack_slow(a2s)
    return _final(g2, a2f[:N], a2s[:N], dinv_col, b2_row)


# speedup vs baseline: 1.1318x; 1.1318x over previous
import functools

import jax
import jax.numpy as jnp
from jax import lax
from jax.experimental import pallas as pl
from jax.experimental.pallas import tpu as pltpu
from jax.experimental.pallas import tpu_sc as plsc

N = 10000
D = 128
E = 320000

NW = 32
K = 128
CHUNKS = 79
EPW = K * CHUNKS
EPAD = EPW * NW
NPAD = 10112
NROWCH = NPAD // K
NPS = NPAD // 16

CH_A = 146
CH_B = 12
MAXCH = max(CH_A, CH_B)

_mesh = plsc.VectorSubcoreMesh(core_axis_name="c", subcore_axis_name="s")


@functools.partial(
    pl.kernel,
    out_type=jax.ShapeDtypeStruct((2 * NPAD,), jnp.float32),
    mesh=_mesh,
    scratch_types=[
        pltpu.VMEM((K,), jnp.int32),
        pltpu.VMEM((K,), jnp.int32),
        pltpu.VMEM((K,), jnp.float32),
        pltpu.VMEM((NPS,), jnp.float32),
        pltpu.VMEM_SHARED((NPAD,), jnp.float32),
        pltpu.SemaphoreType.DMA,
        pltpu.SemaphoreType.DMA,
    ],
)
def _deg_kernel(dst_hbm, out_hbm, dst_v0, dst_v1, ones_v, stage_v, acc_sh,
                dsem0, dsem1):
    c = lax.axis_index("c")
    s = lax.axis_index("s")
    wid = s * 2 + c

    ones16 = jnp.ones((16,), jnp.float32)
    zeros16 = jnp.zeros((16,), jnp.float32)

    def _init(i, _):
        ones_v[pl.ds(i * 16, 16)] = ones16
        return 0

    lax.fori_loop(0, K // 16, _init, 0)

    def _zero(i, _):
        stage_v[pl.ds(i * 16, 16)] = zeros16
        return 0

    lax.fori_loop(0, NPS // 16, _zero, 0)

    pltpu.sync_copy(stage_v, acc_sh.at[pl.ds(s * NPS, NPS)])
    plsc.subcore_barrier()

    base = wid * EPW
    dbufs = ((dst_v0, dsem0), (dst_v1, dsem1))

    def _dissue(i, b):
        dstv, dsem = dbufs[b]
        pltpu.async_copy(dst_hbm.at[pl.ds(base + i * K, K)], dstv, dsem)

    def _dconsume(i, b):
        dstv, dsem = dbufs[b]
        pltpu.make_async_copy(dst_hbm.at[pl.ds(base + i * K, K)], dstv,
                              dsem).wait()
        pltpu.sync_copy(ones_v, acc_sh.at[dstv], add=True)

    _dissue(0, 0)

    def _body(t, _):
        for b in range(2):
            i = 2 * t + b

            @pl.when(i + 1 < CHUNKS)
            def _():
                _dissue(i + 1, 1 - b)

            @pl.when(i < CHUNKS)
            def _():
                _dconsume(i, b)

        return 0

    lax.fori_loop(0, (CHUNKS + 1) // 2, _body, 0)
    plsc.subcore_barrier()

    pltpu.sync_copy(acc_sh.at[pl.ds(s * NPS, NPS)], stage_v)
    pltpu.sync_copy(stage_v, out_hbm.at[pl.ds(c * NPAD + s * NPS, NPS)])


@functools.partial(
    pl.kernel,
    out_type=jax.ShapeDtypeStruct((2 * NPAD, D), jnp.float32),
    mesh=_mesh,
    scratch_types=[
        pltpu.VMEM((K,), jnp.int32),
        pltpu.VMEM((K,), jnp.int32),
        pltpu.VMEM((K,), jnp.int32),
        pltpu.VMEM((K,), jnp.int32),
        pltpu.VMEM((K,), jnp.int32),
        pltpu.VMEM((K,), jnp.int32),
        pltpu.VMEM((K, D), jnp.float32),
        pltpu.VMEM((K, D), jnp.float32),
        pltpu.VMEM((K, D), jnp.float32),
        pltpu.VMEM_SHARED((NPAD, D), jnp.float32),
        pltpu.SemaphoreType.DMA,
        pltpu.SemaphoreType.DMA,
        pltpu.SemaphoreType.DMA,
        pltpu.SemaphoreType.DMA,
        pltpu.SemaphoreType.DMA,
        pltpu.SemaphoreType.DMA,
    ],
)
def _scatter_kernel(g_hbm, src_hbm, dst_hbm, out_hbm,
                    srcv0, srcv1, srcv2, dstv0, dstv1, dstv2,
                    rows0, rows1, rows2, acc_sh,
                    sem0, sem1, sem2, dsem0, dsem1, dsem2):
    c = lax.axis_index("c")
    s = lax.axis_index("s")

    zeros16 = jnp.zeros((16,), jnp.float32)

    def _zrow(i, _):
        for j in range(D // 16):
            rows0[i, pl.ds(j * 16, 16)] = zeros16
        return 0

    lax.fori_loop(0, K, _zrow, 0)

    nch = jnp.where(c == 0, CH_A, CH_B)
    base = jnp.where(c == 0, s * CH_A, 16 * CH_A + s * CH_B) * K

    NZT = (NROWCH + 15) // 16
    for t in range(NZT):
        cc = s + 16 * t

        @pl.when(cc < NROWCH)
        def _():
            pltpu.async_copy(rows0, acc_sh.at[pl.ds(cc * K, K)], sem0)

    for t in range(NZT):
        cc = s + 16 * t

        @pl.when(cc < NROWCH)
        def _():
            pltpu.make_async_copy(rows0, acc_sh.at[pl.ds(cc * K, K)],
                                  sem0).wait()

    plsc.subcore_barrier()

    bufs = ((rows0, sem0, srcv0, dstv0, dsem0),
            (rows1, sem1, srcv1, dstv1, dsem1),
            (rows2, sem2, srcv2, dstv2, dsem2))
    NB = len(bufs)

    def _issue_idx(i, b):
        _, _, srcv, dstv, dsem = bufs[b]
        pltpu.async_copy(src_hbm.at[pl.ds(base + i * K, K)], srcv, dsem)
        pltpu.async_copy(dst_hbm.at[pl.ds(base + i * K, K)], dstv, dsem)

    def _issue_gather(i, b):
        rows, sem, srcv, dstv, dsem = bufs[b]
        pltpu.make_async_copy(src_hbm.at[pl.ds(base + i * K, K)], srcv,
                              dsem).wait()
        pltpu.make_async_copy(dst_hbm.at[pl.ds(base + i * K, K)], dstv,
                              dsem).wait()
        pltpu.async_copy(g_hbm.at[srcv], rows, sem)

    def _consume(i, b):
        rows, sem, srcv, dstv, dsem = bufs[b]
        pltpu.make_async_copy(g_hbm.at[srcv], rows, sem).wait()
        pltpu.sync_copy(rows, acc_sh.at[dstv], add=True)

    _issue_idx(0, 0)
    _issue_idx(1, 1)
    _issue_gather(0, 0)

    def _body(t, _):
        for b in range(NB):
            i = NB * t + b

            @pl.when(i + 2 < nch)
            def _():
                _issue_idx(i + 2, (b + 2) % NB)

            @pl.when(i + 1 < nch)
            def _():
                _issue_gather(i + 1, (b + 1) % NB)

            @pl.when(i < nch)
            def _():
                _consume(i, b)

        return 0

    lax.fori_loop(0, (MAXCH + NB - 1) // NB, _body, 0)
    plsc.subcore_barrier()

    rbufs = ((rows0, sem0), (rows1, sem1), (rows2, sem2))

    def _hbm_write(t):
        cc = s + 16 * t

        @pl.when(cc < NROWCH)
        def _():
            b, wsem = rbufs[t % 3]
            pltpu.async_copy(acc_sh.at[pl.ds(cc * K, K)], b, dsem0)
            pltpu.make_async_copy(acc_sh.at[pl.ds(cc * K, K)], b,
                                  dsem0).wait()
            pltpu.async_copy(b, out_hbm.at[pl.ds(c * NPAD + cc * K, K)],
                             wsem)

    def _hbm_drain(t):
        cc = s + 16 * t

        @pl.when(cc < NROWCH)
        def _():
            b, wsem = rbufs[t % 3]
            pltpu.make_async_copy(
                b, out_hbm.at[pl.ds(c * NPAD + cc * K, K)], wsem).wait()

    NCT = (NROWCH + 15) // 16
    for t in range(NCT):
        if t >= 3:
            _hbm_drain(t - 3)
        _hbm_write(t)
    for t in range(max(0, NCT - 3), NCT):
        _hbm_drain(t)


_RB = 400
_GRID = N // _RB


def _mm_scale_body(x_ref, wt_ref, dinv_ref, o_ref):
    o_ref[...] = (
        jnp.dot(x_ref[...], wt_ref[...], preferred_element_type=jnp.float32)
        * dinv_ref[...]
    )


def _layer_body(g_ref, a0_ref, a1_ref, dinv_ref, b_ref, wt_ref, o_ref):
    h = (g_ref[...] + a0_ref[...] + a1_ref[...]) * dinv_ref[...] + b_ref[...]
    h = jnp.maximum(h, 0.0)
    o_ref[...] = (
        jnp.dot(h, wt_ref[...], preferred_element_type=jnp.float32)
        * dinv_ref[...]
    )


def _final_body(g_ref, a0_ref, a1_ref, dinv_ref, b_ref, o_ref):
    o_ref[...] = (
        (g_ref[...] + a0_ref[...] + a1_ref[...]) * dinv_ref[...] + b_ref[...]
    )


_row_spec = pl.BlockSpec((_RB, D), lambda i: (i, 0))
_w_spec = pl.BlockSpec((D, D), lambda i: (0, 0))
_dinv_spec = pl.BlockSpec((_RB, 1), lambda i: (i, 0))
_b_spec = pl.BlockSpec((1, D), lambda i: (0, 0))
_node_shape = jax.ShapeDtypeStruct((N, D), jnp.float32)


def _mm_scale(x, wt, dinv_col):
    return pl.pallas_call(
        _mm_scale_body,
        grid=(_GRID,),
        in_specs=[_row_spec, _w_spec, _dinv_spec],
        out_specs=_row_spec,
        out_shape=_node_shape,
    )(x, wt, dinv_col)


def _layer(g, a0, a1, dinv_col, b_row, wt):
    return pl.pallas_call(
        _layer_body,
        grid=(_GRID,),
        in_specs=[_row_spec, _row_spec, _row_spec, _dinv_spec, _b_spec,
                  _w_spec],
        out_specs=_row_spec,
        out_shape=_node_shape,
    )(g, a0, a1, dinv_col, b_row, wt)


def _final(g, a0, a1, dinv_col, b_row):
    return pl.pallas_call(
        _final_body,
        grid=(_GRID,),
        in_specs=[_row_spec, _row_spec, _row_spec, _dinv_spec, _b_spec],
        out_specs=_row_spec,
        out_shape=_node_shape,
    )(g, a0, a1, dinv_col, b_row)


def kernel(x, edge_index, W1, b1, W2, b2):
    src = edge_index[0].astype(jnp.int32)
    dst = edge_index[1].astype(jnp.int32)
    pad = EPAD - E
    src_p = jnp.concatenate([src, jnp.zeros((pad,), jnp.int32)])
    dst_p = jnp.concatenate([dst, jnp.full((pad,), N, jnp.int32)])

    degp = _deg_kernel(dst_p)
    deg = degp[:N] + degp[NPAD:NPAD + N] + 1.0
    dinv_col = lax.rsqrt(deg)[:, None]
    b1_row = b1[None, :]
    b2_row = b2[None, :]

    g1 = _mm_scale(x, W1.T, dinv_col)
    a1 = _scatter_kernel(g1, src_p, dst_p)
    g2 = _layer(g1, a1[:N], a1[NPAD:NPAD + N], dinv_col, b1_row, W2.T)
    a2 = _scatter_kernel(g2, src_p, dst_p)
    return _final(g2, a2[:N], a2[NPAD:NPAD + N], dinv_col, b2_row)
